# baseline (speedup 1.0000x reference)
import jax
import jax.numpy as jnp
from jax import lax
from jax.experimental import pallas as pl

_VOCAB = 1000
_EMB = 128
_BATCH = 16384
_BM = 2048


def _tc_fused_kernel(x_ref, t_ref, wt_ref, b_ref, o_ref, e_ref):
    xb = x_ref[...].astype(jnp.int16)
    iota = lax.broadcasted_iota(jnp.int16, (_BM, _VOCAB), 1)
    oh = (xb == iota).astype(jnp.bfloat16)
    emb = jnp.dot(oh, t_ref[...], preferred_element_type=jnp.float32)
    e_ref[...] = emb
    o_ref[...] = (
        jnp.dot(emb.astype(jnp.bfloat16), wt_ref[...],
                preferred_element_type=jnp.float32)
        + b_ref[0:1, :]
    )


@jax.jit
def kernel(x, table, W, b):
    xi = x.astype(jnp.int32)
    out, emb = pl.pallas_call(
        _tc_fused_kernel,
        grid=(_BATCH // _BM,),
        in_specs=[
            pl.BlockSpec((_BM, 1), lambda i: (i, 0)),
            pl.BlockSpec((_VOCAB, _EMB), lambda i: (0, 0)),
            pl.BlockSpec((_EMB, _VOCAB), lambda i: (0, 0)),
            pl.BlockSpec((1, _VOCAB), lambda i: (0, 0)),
        ],
        out_specs=[pl.BlockSpec((_BM, _VOCAB), lambda i: (i, 0)),
                   pl.BlockSpec((_BM, _EMB), lambda i: (i, 0))],
        out_shape=[jax.ShapeDtypeStruct((_BATCH, _VOCAB), jnp.float32),
                   jax.ShapeDtypeStruct((_BATCH, _EMB), jnp.float32)],
    )(xi.reshape(_BATCH, 1), table.astype(jnp.bfloat16),
      W.T.astype(jnp.bfloat16), b.reshape(1, _VOCAB))
    return out, emb
